# baseline (speedup 1.0000x reference)
import jax
import jax.numpy as jnp
from jax.experimental import pallas as pl
from jax.experimental.pallas import tpu as pltpu

TOKEN_DIM = 768
INNER = 512
PLACEHOLDER_TOKEN = 42
N_TOKENS = 8192
BLOCK = 2048


def _body(tok_ref, emb_ref, lv_any, wv2_any, wo2_any, bo2_any, wnet_any,
          bnet_any, out_ref, tokcol_ref, ph_ref, lv_v, wv2_v, wo2_v, bo2_v,
          wnet_v, bnet_v, sems):
    i = pl.program_id(0)

    @pl.when(i == 0)
    def _prologue():
        copies = [
            pltpu.make_async_copy(lv_any, lv_v, sems.at[0]),
            pltpu.make_async_copy(wv2_any, wv2_v, sems.at[1]),
            pltpu.make_async_copy(wo2_any, wo2_v, sems.at[2]),
            pltpu.make_async_copy(bo2_any, bo2_v, sems.at[3]),
            pltpu.make_async_copy(wnet_any, wnet_v, sems.at[4]),
            pltpu.make_async_copy(bnet_any, bnet_v, sems.at[5]),
        ]
        for cp in copies:
            cp.start()
        tokcol_ref[...] = jnp.swapaxes(tok_ref[...], 0, 1)
        for cp in copies:
            cp.wait()
        x = lv_v[...]
        v = jnp.dot(x, wv2_v[...], preferred_element_type=jnp.float32)
        x2 = jnp.dot(v, wo2_v[...], preferred_element_type=jnp.float32)
        x2 = x2 + bo2_v[...]
        ph = jnp.dot(x2, wnet_v[...], preferred_element_type=jnp.float32)
        ph_ref[...] = ph + bnet_v[...]

    mask = tokcol_ref[pl.ds(i * BLOCK, BLOCK), :] == PLACEHOLDER_TOKEN
    out_ref[...] = jnp.where(mask, ph_ref[...], emb_ref[...])


def kernel(tokenized_text, embedded_text, image_embeds, learnable_vector,
           Wq1, Wk1, Wv1, Wo1, bo1, Wq2, Wk2, Wv2, Wo2, bo2, Wnet, bnet):
    emb = embedded_text.reshape(N_TOKENS, TOKEN_DIM)
    lv = learnable_vector.reshape(1, TOKEN_DIM)
    out = pl.pallas_call(
        _body,
        grid=(N_TOKENS // BLOCK,),
        in_specs=[
            pl.BlockSpec((1, N_TOKENS), lambda i: (0, 0)),
            pl.BlockSpec((BLOCK, TOKEN_DIM), lambda i: (i, 0)),
            pl.BlockSpec(memory_space=pl.ANY),
            pl.BlockSpec(memory_space=pl.ANY),
            pl.BlockSpec(memory_space=pl.ANY),
            pl.BlockSpec(memory_space=pl.ANY),
            pl.BlockSpec(memory_space=pl.ANY),
            pl.BlockSpec(memory_space=pl.ANY),
        ],
        out_specs=pl.BlockSpec((BLOCK, TOKEN_DIM), lambda i: (i, 0)),
        out_shape=jax.ShapeDtypeStruct((N_TOKENS, TOKEN_DIM), jnp.float32),
        scratch_shapes=[
            pltpu.VMEM((N_TOKENS, 1), jnp.int32),
            pltpu.VMEM((1, TOKEN_DIM), jnp.float32),
            pltpu.VMEM((1, TOKEN_DIM), jnp.float32),
            pltpu.VMEM((TOKEN_DIM, INNER), jnp.float32),
            pltpu.VMEM((INNER, TOKEN_DIM), jnp.float32),
            pltpu.VMEM((1, TOKEN_DIM), jnp.float32),
            pltpu.VMEM((TOKEN_DIM, TOKEN_DIM), jnp.float32),
            pltpu.VMEM((1, TOKEN_DIM), jnp.float32),
            pltpu.SemaphoreType.DMA((6,)),
        ],
        compiler_params=pltpu.CompilerParams(
            dimension_semantics=("arbitrary",)),
    )(tokenized_text, emb, lv, Wv2, Wo2, bo2.reshape(1, TOKEN_DIM), Wnet,
      bnet.reshape(1, TOKEN_DIM))
    return out.reshape(1, N_TOKENS, TOKEN_DIM)
